# baseline (speedup 1.0000x reference)
import functools

import jax
import jax.numpy as jnp
from jax import lax
from jax.experimental import pallas as pl
from jax.experimental.pallas import tpu as pltpu
from jax.experimental.pallas import tpu_sc as plsc

_NC = 2
_NS = 16
_NW = _NC * _NS
_LANES = 16

_CHUNK = 1000
_NBUF = 1


def _sc_segment_sum(y, srcp, dstp, zeros, n_pad, kch):
    mesh = plsc.VectorSubcoreMesh(core_axis_name="c", subcore_axis_name="s")
    rows_per_tile = n_pad // _NS

    @functools.partial(
        pl.kernel,
        mesh=mesh,
        compiler_params=pltpu.CompilerParams(use_tc_tiling_on_sc=False,
                                             disable_bounds_checks=True),
        out_type=jax.ShapeDtypeStruct((_NC, n_pad, _LANES), jnp.float32),
        scratch_types=[
            pltpu.VMEM((kch, _CHUNK), jnp.int32),
            pltpu.VMEM((kch, _CHUNK), jnp.int32),
            pltpu.VMEM((_NBUF, _CHUNK, _LANES), jnp.float32),
            pltpu.VMEM_SHARED((n_pad, _LANES), jnp.float32),
            pltpu.VMEM_SHARED((n_pad, _LANES), jnp.float32),
            pltpu.SemaphoreType.DMA,
        ],
    )
    def k(y_hbm, srcp_hbm, dstp_hbm, zeros_hbm, out_hbm,
          src_v, dst_v, buf_v, acc_sh, table_sh, gsem):
        cid = lax.axis_index("c")
        sid = lax.axis_index("s")
        wid = cid * _NS + sid

        row0 = sid * rows_per_tile
        pltpu.sync_copy(zeros_hbm.at[pl.ds(row0, rows_per_tile)],
                        acc_sh.at[pl.ds(row0, rows_per_tile)])
        pltpu.sync_copy(y_hbm.at[pl.ds(row0, rows_per_tile)],
                        table_sh.at[pl.ds(row0, rows_per_tile)])
        pltpu.sync_copy(srcp_hbm.at[wid], src_v)
        pltpu.sync_copy(dstp_hbm.at[wid], dst_v)
        plsc.subcore_barrier()

        for j in range(kch):
            pltpu.sync_copy(table_sh.at[src_v.at[j]], buf_v.at[0])
            pltpu.sync_copy(buf_v.at[0], acc_sh.at[dst_v.at[j]], add=True)
        plsc.subcore_barrier()

        pltpu.sync_copy(acc_sh.at[pl.ds(row0, rows_per_tile)],
                        out_hbm.at[cid, pl.ds(row0, rows_per_tile)])

    return k(y, srcp, dstp, zeros)


def _tc_input(x, w, n_pad):
    n = x.shape[0]
    def body(x_ref, w_ref, rel_ref, root_ref):
        y = jnp.dot(x_ref[...], w_ref[...], preferred_element_type=jnp.float32)
        rel_ref[pl.ds(0, n), :] = y[:, :_LANES]
        root_ref[...] = y[:, _LANES:]

    return pl.pallas_call(
        body,
        out_shape=(
            jax.ShapeDtypeStruct((n_pad, _LANES), jnp.float32),
            jax.ShapeDtypeStruct((n, _LANES), jnp.float32),
        ),
    )(x, w)


def _tc_mid(p1, y_root, b1, w2rel_t, w2root_t, n_pad):
    n = y_root.shape[0]
    def body(p1_ref, yr_ref, b1_ref, wrel_ref, wroot_ref,
             rel_ref, root_ref):
        h = p1_ref[0, pl.ds(0, n), :] + p1_ref[1, pl.ds(0, n), :] \
            + yr_ref[...] + b1_ref[...]
        h = jnp.maximum(h, 0.0)
        rel_ref[pl.ds(0, n), :] = jnp.dot(h, wrel_ref[...],
                                          preferred_element_type=jnp.float32)
        root_ref[...] = jnp.dot(h, wroot_ref[...],
                                preferred_element_type=jnp.float32)

    return pl.pallas_call(
        body,
        out_shape=(
            jax.ShapeDtypeStruct((n_pad, _LANES), jnp.float32),
            jax.ShapeDtypeStruct((n, _LANES), jnp.float32),
        ),
    )(p1, y_root, b1, w2rel_t, w2root_t)


def _tc_out(p2, y_root, b2, c, n_pad):
    n = y_root.shape[0]

    def body(p2_ref, yr_ref, b2_ref, out_ref):
        o = p2_ref[0, pl.ds(0, n), :] + p2_ref[1, pl.ds(0, n), :] \
            + yr_ref[...] + b2_ref[...]
        col = lax.broadcasted_iota(jnp.int32, o.shape, 1)
        mask = col < c
        neg = jnp.float32(-1e30)
        om = jnp.where(mask, o, neg)
        m = jnp.max(om, axis=1, keepdims=True)
        e = jnp.where(mask, jnp.exp(o - m), 0.0)
        s = jnp.sum(e, axis=1, keepdims=True)
        out_ref[...] = ((o - m) - jnp.log(s))[:, :c]

    return pl.pallas_call(
        body,
        out_shape=jax.ShapeDtypeStruct((n, c), jnp.float32),
    )(p2, y_root, b2)


def kernel(x, edge_index, W1_rel, b1_rel, W1_root, W2_rel, b2_rel, W2_root):
    n, d = x.shape
    h_dim = W1_rel.shape[0]
    c = W2_rel.shape[0]
    e = edge_index.shape[1]

    kch = -(-e // (_NW * _CHUNK))
    e_pad = _NW * kch * _CHUNK
    n_pad = -(-(n + 1) // 128) * 128

    src = edge_index[0]
    dst = edge_index[1]
    pad = e_pad - e
    if pad:
        src = jnp.concatenate([src, jnp.zeros((pad,), jnp.int32)])
        trash = n + jnp.arange(pad, dtype=jnp.int32) % (n_pad - n)
        dst = jnp.concatenate([dst, trash])
    srcp = src.reshape(_NW, kch, _CHUNK)
    dstp = dst.reshape(_NW, kch, _CHUNK)
    zeros = jnp.zeros((n_pad, _LANES), jnp.float32)

    w1 = jnp.concatenate([W1_rel, W1_root], axis=0).T
    y_rel, y_root = _tc_input(x, w1, n_pad)

    p1 = _sc_segment_sum(y_rel, srcp, dstp, zeros, n_pad, kch)

    w2rel_t = jnp.zeros((h_dim, _LANES), jnp.float32).at[:, :c].set(W2_rel.T)
    w2root_t = jnp.zeros((h_dim, _LANES), jnp.float32).at[:, :c].set(W2_root.T)
    b1 = b1_rel.reshape(1, h_dim)
    y2_rel, y2_root = _tc_mid(p1, y_root, b1, w2rel_t, w2root_t, n_pad)

    p2 = _sc_segment_sum(y2_rel, srcp, dstp, zeros, n_pad, kch)

    b2 = jnp.zeros((1, _LANES), jnp.float32).at[0, :c].set(b2_rel)
    return _tc_out(p2, y2_root, b2, c, n_pad)


# speedup vs baseline: 1.0279x; 1.0279x over previous
import functools

import jax
import jax.numpy as jnp
from jax import lax
from jax.experimental import pallas as pl
from jax.experimental.pallas import tpu as pltpu
from jax.experimental.pallas import tpu_sc as plsc

_NC = 2
_NS = 16
_NW = _NC * _NS
_LANES = 16

_CHUNK = 2500
_NBUF = 1


def _sc_segment_sum(y, srcp, dstp, zeros, n_pad, kch):
    mesh = plsc.VectorSubcoreMesh(core_axis_name="c", subcore_axis_name="s")
    rows_per_tile = n_pad // _NS

    @functools.partial(
        pl.kernel,
        mesh=mesh,
        compiler_params=pltpu.CompilerParams(use_tc_tiling_on_sc=False,
                                             disable_bounds_checks=True),
        out_type=jax.ShapeDtypeStruct((_NC, n_pad, _LANES), jnp.float32),
        scratch_types=[
            pltpu.VMEM((kch, _CHUNK), jnp.int32),
            pltpu.VMEM((kch, _CHUNK), jnp.int32),
            pltpu.VMEM((_NBUF, _CHUNK, _LANES), jnp.float32),
            pltpu.VMEM_SHARED((n_pad, _LANES), jnp.float32),
            pltpu.VMEM_SHARED((n_pad, _LANES), jnp.float32),
            pltpu.SemaphoreType.DMA,
        ],
    )
    def k(y_hbm, srcp_hbm, dstp_hbm, zeros_hbm, out_hbm,
          src_v, dst_v, buf_v, acc_sh, table_sh, gsem):
        cid = lax.axis_index("c")
        sid = lax.axis_index("s")
        wid = cid * _NS + sid

        row0 = sid * rows_per_tile
        pltpu.sync_copy(zeros_hbm.at[pl.ds(row0, rows_per_tile)],
                        acc_sh.at[pl.ds(row0, rows_per_tile)])
        pltpu.sync_copy(y_hbm.at[pl.ds(row0, rows_per_tile)],
                        table_sh.at[pl.ds(row0, rows_per_tile)])
        pltpu.sync_copy(srcp_hbm.at[wid], src_v)
        pltpu.sync_copy(dstp_hbm.at[wid], dst_v)
        plsc.subcore_barrier()

        for j in range(kch):
            pltpu.sync_copy(table_sh.at[src_v.at[j]], buf_v.at[0])
            pltpu.sync_copy(buf_v.at[0], acc_sh.at[dst_v.at[j]], add=True)
        plsc.subcore_barrier()

        pltpu.sync_copy(acc_sh.at[pl.ds(row0, rows_per_tile)],
                        out_hbm.at[cid, pl.ds(row0, rows_per_tile)])

    return k(y, srcp, dstp, zeros)


def _tc_input(x, w, n_pad):
    n = x.shape[0]
    def body(x_ref, w_ref, rel_ref, root_ref):
        y = jnp.dot(x_ref[...], w_ref[...], preferred_element_type=jnp.float32)
        rel_ref[pl.ds(0, n), :] = y[:, :_LANES]
        root_ref[...] = y[:, _LANES:]

    return pl.pallas_call(
        body,
        out_shape=(
            jax.ShapeDtypeStruct((n_pad, _LANES), jnp.float32),
            jax.ShapeDtypeStruct((n, _LANES), jnp.float32),
        ),
    )(x, w)


def _tc_mid(p1, y_root, b1, w2rel_t, w2root_t, n_pad):
    n = y_root.shape[0]
    def body(p1_ref, yr_ref, b1_ref, wrel_ref, wroot_ref,
             rel_ref, root_ref):
        h = p1_ref[0, pl.ds(0, n), :] + p1_ref[1, pl.ds(0, n), :] \
            + yr_ref[...] + b1_ref[...]
        h = jnp.maximum(h, 0.0)
        rel_ref[pl.ds(0, n), :] = jnp.dot(h, wrel_ref[...],
                                          preferred_element_type=jnp.float32)
        root_ref[...] = jnp.dot(h, wroot_ref[...],
                                preferred_element_type=jnp.float32)

    return pl.pallas_call(
        body,
        out_shape=(
            jax.ShapeDtypeStruct((n_pad, _LANES), jnp.float32),
            jax.ShapeDtypeStruct((n, _LANES), jnp.float32),
        ),
    )(p1, y_root, b1, w2rel_t, w2root_t)


def _tc_out(p2, y_root, b2, c, n_pad):
    n = y_root.shape[0]

    def body(p2_ref, yr_ref, b2_ref, out_ref):
        o = p2_ref[0, pl.ds(0, n), :] + p2_ref[1, pl.ds(0, n), :] \
            + yr_ref[...] + b2_ref[...]
        col = lax.broadcasted_iota(jnp.int32, o.shape, 1)
        mask = col < c
        neg = jnp.float32(-1e30)
        om = jnp.where(mask, o, neg)
        m = jnp.max(om, axis=1, keepdims=True)
        e = jnp.where(mask, jnp.exp(o - m), 0.0)
        s = jnp.sum(e, axis=1, keepdims=True)
        out_ref[...] = ((o - m) - jnp.log(s))[:, :c]

    return pl.pallas_call(
        body,
        out_shape=jax.ShapeDtypeStruct((n, c), jnp.float32),
    )(p2, y_root, b2)


def kernel(x, edge_index, W1_rel, b1_rel, W1_root, W2_rel, b2_rel, W2_root):
    n, d = x.shape
    h_dim = W1_rel.shape[0]
    c = W2_rel.shape[0]
    e = edge_index.shape[1]

    kch = -(-e // (_NW * _CHUNK))
    e_pad = _NW * kch * _CHUNK
    n_pad = -(-(n + 1) // 128) * 128

    src = edge_index[0]
    dst = edge_index[1]
    pad = e_pad - e
    if pad:
        src = jnp.concatenate([src, jnp.zeros((pad,), jnp.int32)])
        trash = n + jnp.arange(pad, dtype=jnp.int32) % (n_pad - n)
        dst = jnp.concatenate([dst, trash])
    srcp = src.reshape(_NW, kch, _CHUNK)
    dstp = dst.reshape(_NW, kch, _CHUNK)
    zeros = jnp.zeros((n_pad, _LANES), jnp.float32)

    w1 = jnp.concatenate([W1_rel, W1_root], axis=0).T
    y_rel, y_root = _tc_input(x, w1, n_pad)

    p1 = _sc_segment_sum(y_rel, srcp, dstp, zeros, n_pad, kch)

    w2rel_t = jnp.zeros((h_dim, _LANES), jnp.float32).at[:, :c].set(W2_rel.T)
    w2root_t = jnp.zeros((h_dim, _LANES), jnp.float32).at[:, :c].set(W2_root.T)
    b1 = b1_rel.reshape(1, h_dim)
    y2_rel, y2_root = _tc_mid(p1, y_root, b1, w2rel_t, w2root_t, n_pad)

    p2 = _sc_segment_sum(y2_rel, srcp, dstp, zeros, n_pad, kch)

    b2 = jnp.zeros((1, _LANES), jnp.float32).at[0, :c].set(b2_rel)
    return _tc_out(p2, y2_root, b2, c, n_pad)


# speedup vs baseline: 1.0909x; 1.0613x over previous
import functools

import jax
import jax.numpy as jnp
from jax import lax
from jax.experimental import pallas as pl
from jax.experimental.pallas import tpu as pltpu
from jax.experimental.pallas import tpu_sc as plsc

_NC = 2
_NS = 16
_NW = _NC * _NS
_LANES = 16

_CHUNK = 5000
_NBUF = 1


def _sc_segment_sum(y, srcp, dstp, zeros, n_pad, kch):
    mesh = plsc.VectorSubcoreMesh(core_axis_name="c", subcore_axis_name="s")
    rows_per_tile = n_pad // _NS

    @functools.partial(
        pl.kernel,
        mesh=mesh,
        compiler_params=pltpu.CompilerParams(use_tc_tiling_on_sc=False,
                                             disable_bounds_checks=True),
        out_type=jax.ShapeDtypeStruct((_NC, n_pad, _LANES), jnp.float32),
        scratch_types=[
            pltpu.VMEM((kch, _CHUNK), jnp.int32),
            pltpu.VMEM((kch, _CHUNK), jnp.int32),
            pltpu.VMEM((_NBUF, _CHUNK, _LANES), jnp.float32),
            pltpu.VMEM_SHARED((n_pad, _LANES), jnp.float32),
            pltpu.VMEM_SHARED((n_pad, _LANES), jnp.float32),
            pltpu.SemaphoreType.DMA,
        ],
    )
    def k(y_hbm, srcp_hbm, dstp_hbm, zeros_hbm, out_hbm,
          src_v, dst_v, buf_v, acc_sh, table_sh, gsem):
        cid = lax.axis_index("c")
        sid = lax.axis_index("s")
        wid = cid * _NS + sid

        row0 = sid * rows_per_tile
        pltpu.sync_copy(zeros_hbm.at[pl.ds(row0, rows_per_tile)],
                        acc_sh.at[pl.ds(row0, rows_per_tile)])
        pltpu.sync_copy(y_hbm.at[pl.ds(row0, rows_per_tile)],
                        table_sh.at[pl.ds(row0, rows_per_tile)])
        pltpu.sync_copy(srcp_hbm.at[wid], src_v)
        pltpu.sync_copy(dstp_hbm.at[wid], dst_v)
        plsc.subcore_barrier()

        for j in range(kch):
            pltpu.sync_copy(table_sh.at[src_v.at[j]], buf_v.at[0])
            pltpu.sync_copy(buf_v.at[0], acc_sh.at[dst_v.at[j]], add=True)
        plsc.subcore_barrier()

        pltpu.sync_copy(acc_sh.at[pl.ds(row0, rows_per_tile)],
                        out_hbm.at[cid, pl.ds(row0, rows_per_tile)])

    return k(y, srcp, dstp, zeros)


def _tc_input(x, w, n_pad):
    n = x.shape[0]
    def body(x_ref, w_ref, rel_ref, root_ref):
        y = jnp.dot(x_ref[...], w_ref[...], preferred_element_type=jnp.float32)
        rel_ref[pl.ds(0, n), :] = y[:, :_LANES]
        root_ref[...] = y[:, _LANES:]

    return pl.pallas_call(
        body,
        out_shape=(
            jax.ShapeDtypeStruct((n_pad, _LANES), jnp.float32),
            jax.ShapeDtypeStruct((n, _LANES), jnp.float32),
        ),
    )(x, w)


def _tc_mid(p1, y_root, b1, w2rel_t, w2root_t, n_pad):
    n = y_root.shape[0]
    def body(p1_ref, yr_ref, b1_ref, wrel_ref, wroot_ref,
             rel_ref, root_ref):
        h = p1_ref[0, pl.ds(0, n), :] + p1_ref[1, pl.ds(0, n), :] \
            + yr_ref[...] + b1_ref[...]
        h = jnp.maximum(h, 0.0)
        rel_ref[pl.ds(0, n), :] = jnp.dot(h, wrel_ref[...],
                                          preferred_element_type=jnp.float32)
        root_ref[...] = jnp.dot(h, wroot_ref[...],
                                preferred_element_type=jnp.float32)

    return pl.pallas_call(
        body,
        out_shape=(
            jax.ShapeDtypeStruct((n_pad, _LANES), jnp.float32),
            jax.ShapeDtypeStruct((n, _LANES), jnp.float32),
        ),
    )(p1, y_root, b1, w2rel_t, w2root_t)


def _tc_out(p2, y_root, b2, c, n_pad):
    n = y_root.shape[0]

    def body(p2_ref, yr_ref, b2_ref, out_ref):
        o = p2_ref[0, pl.ds(0, n), :] + p2_ref[1, pl.ds(0, n), :] \
            + yr_ref[...] + b2_ref[...]
        col = lax.broadcasted_iota(jnp.int32, o.shape, 1)
        mask = col < c
        neg = jnp.float32(-1e30)
        om = jnp.where(mask, o, neg)
        m = jnp.max(om, axis=1, keepdims=True)
        e = jnp.where(mask, jnp.exp(o - m), 0.0)
        s = jnp.sum(e, axis=1, keepdims=True)
        out_ref[...] = ((o - m) - jnp.log(s))[:, :c]

    return pl.pallas_call(
        body,
        out_shape=jax.ShapeDtypeStruct((n, c), jnp.float32),
    )(p2, y_root, b2)


def kernel(x, edge_index, W1_rel, b1_rel, W1_root, W2_rel, b2_rel, W2_root):
    n, d = x.shape
    h_dim = W1_rel.shape[0]
    c = W2_rel.shape[0]
    e = edge_index.shape[1]

    kch = -(-e // (_NW * _CHUNK))
    e_pad = _NW * kch * _CHUNK
    n_pad = -(-(n + 1) // 128) * 128

    src = edge_index[0]
    dst = edge_index[1]
    pad = e_pad - e
    if pad:
        src = jnp.concatenate([src, jnp.zeros((pad,), jnp.int32)])
        trash = n + jnp.arange(pad, dtype=jnp.int32) % (n_pad - n)
        dst = jnp.concatenate([dst, trash])
    srcp = src.reshape(_NW, kch, _CHUNK)
    dstp = dst.reshape(_NW, kch, _CHUNK)
    zeros = jnp.zeros((n_pad, _LANES), jnp.float32)

    w1 = jnp.concatenate([W1_rel, W1_root], axis=0).T
    y_rel, y_root = _tc_input(x, w1, n_pad)

    p1 = _sc_segment_sum(y_rel, srcp, dstp, zeros, n_pad, kch)

    w2rel_t = jnp.zeros((h_dim, _LANES), jnp.float32).at[:, :c].set(W2_rel.T)
    w2root_t = jnp.zeros((h_dim, _LANES), jnp.float32).at[:, :c].set(W2_root.T)
    b1 = b1_rel.reshape(1, h_dim)
    y2_rel, y2_root = _tc_mid(p1, y_root, b1, w2rel_t, w2root_t, n_pad)

    p2 = _sc_segment_sum(y2_rel, srcp, dstp, zeros, n_pad, kch)

    b2 = jnp.zeros((1, _LANES), jnp.float32).at[0, :c].set(b2_rel)
    return _tc_out(p2, y2_root, b2, c, n_pad)


# speedup vs baseline: 1.1814x; 1.0829x over previous
import functools

import jax
import jax.numpy as jnp
from jax import lax
from jax.experimental import pallas as pl
from jax.experimental.pallas import tpu as pltpu
from jax.experimental.pallas import tpu_sc as plsc

_NC = 2
_NS = 16
_NW = _NC * _NS
_LANES = 16

_CHUNK = 5000
_NBUF = 1


def _sc_segment_sum(y, srcp, dstp, zeros, n_pad, kch):
    mesh = plsc.VectorSubcoreMesh(core_axis_name="c", subcore_axis_name="s")
    rows_per_tile = n_pad // _NS

    @functools.partial(
        pl.kernel,
        mesh=mesh,
        compiler_params=pltpu.CompilerParams(use_tc_tiling_on_sc=False,
                                             disable_bounds_checks=True),
        out_type=jax.ShapeDtypeStruct((_NC, n_pad, _LANES), jnp.float32),
        scratch_types=[
            pltpu.VMEM((kch, _CHUNK), jnp.int32),
            pltpu.VMEM((kch, _CHUNK), jnp.int32),
            pltpu.VMEM((_NBUF, _CHUNK, _LANES), jnp.float32),
            pltpu.VMEM_SHARED((n_pad, _LANES), jnp.float32),
            pltpu.VMEM_SHARED((n_pad, _LANES), jnp.float32),
            pltpu.SemaphoreType.DMA,
        ],
    )
    def k(y_hbm, srcp_hbm, dstp_hbm, zeros_hbm, out_hbm,
          src_v, dst_v, buf_v, acc_sh, table_sh, gsem):
        cid = lax.axis_index("c")
        sid = lax.axis_index("s")
        wid = cid * _NS + sid

        row0 = sid * rows_per_tile
        pltpu.sync_copy(zeros_hbm.at[pl.ds(row0, rows_per_tile)],
                        acc_sh.at[pl.ds(row0, rows_per_tile)])
        pltpu.sync_copy(y_hbm.at[pl.ds(row0, rows_per_tile)],
                        table_sh.at[pl.ds(row0, rows_per_tile)])
        pltpu.sync_copy(srcp_hbm.at[wid], src_v)
        pltpu.sync_copy(dstp_hbm.at[wid], dst_v)
        plsc.subcore_barrier()

        for j in range(kch):
            pltpu.sync_copy(table_sh.at[src_v.at[j]], buf_v.at[0])
            pltpu.sync_copy(buf_v.at[0], acc_sh.at[dst_v.at[j]], add=True)
        plsc.subcore_barrier()

        pltpu.sync_copy(acc_sh.at[pl.ds(row0, rows_per_tile)],
                        out_hbm.at[cid, pl.ds(row0, rows_per_tile)])

    return k(y, srcp, dstp, zeros)


def _sc_segment_sum_direct(y, edge_index, zeros, n_pad, kch):
    mesh = plsc.VectorSubcoreMesh(core_axis_name="c", subcore_axis_name="s")
    rows_per_tile = n_pad // _NS
    per = kch * _CHUNK

    @functools.partial(
        pl.kernel,
        mesh=mesh,
        compiler_params=pltpu.CompilerParams(use_tc_tiling_on_sc=False,
                                             disable_bounds_checks=True),
        out_type=jax.ShapeDtypeStruct((_NC, n_pad, _LANES), jnp.float32),
        scratch_types=[
            pltpu.VMEM((per,), jnp.int32),
            pltpu.VMEM((per,), jnp.int32),
            pltpu.VMEM((_NBUF, _CHUNK, _LANES), jnp.float32),
            pltpu.VMEM_SHARED((n_pad, _LANES), jnp.float32),
            pltpu.VMEM_SHARED((n_pad, _LANES), jnp.float32),
        ],
    )
    def k(y_hbm, ei_hbm, zeros_hbm, out_hbm,
          src_v, dst_v, buf_v, acc_sh, table_sh):
        cid = lax.axis_index("c")
        sid = lax.axis_index("s")
        wid = cid * _NS + sid

        row0 = sid * rows_per_tile
        pltpu.sync_copy(zeros_hbm.at[pl.ds(row0, rows_per_tile)],
                        acc_sh.at[pl.ds(row0, rows_per_tile)])
        pltpu.sync_copy(y_hbm.at[pl.ds(row0, rows_per_tile)],
                        table_sh.at[pl.ds(row0, rows_per_tile)])
        e0 = pl.multiple_of(wid * per, 8)
        pltpu.sync_copy(ei_hbm.at[0, pl.ds(e0, per)], src_v)
        pltpu.sync_copy(ei_hbm.at[1, pl.ds(e0, per)], dst_v)
        plsc.subcore_barrier()

        for j in range(kch):
            sl = pl.ds(j * _CHUNK, _CHUNK)
            pltpu.sync_copy(table_sh.at[src_v.at[sl]], buf_v.at[0])
            pltpu.sync_copy(buf_v.at[0], acc_sh.at[dst_v.at[sl]], add=True)
        plsc.subcore_barrier()

        pltpu.sync_copy(acc_sh.at[pl.ds(row0, rows_per_tile)],
                        out_hbm.at[cid, pl.ds(row0, rows_per_tile)])

    return k(y, edge_index, zeros)


def _tc_input(x, w, n_pad):
    n = x.shape[0]
    def body(x_ref, w_ref, rel_ref, root_ref):
        y = jnp.dot(x_ref[...], w_ref[...], preferred_element_type=jnp.float32)
        rel_ref[pl.ds(0, n), :] = y[:, :_LANES]
        root_ref[...] = y[:, _LANES:]

    return pl.pallas_call(
        body,
        out_shape=(
            jax.ShapeDtypeStruct((n_pad, _LANES), jnp.float32),
            jax.ShapeDtypeStruct((n, _LANES), jnp.float32),
        ),
    )(x, w)


def _tc_mid(p1, y_root, b1, w2rel_t, w2root_t, n_pad):
    n = y_root.shape[0]
    def body(p1_ref, yr_ref, b1_ref, wrel_ref, wroot_ref,
             rel_ref, root_ref):
        h = p1_ref[0, pl.ds(0, n), :] + p1_ref[1, pl.ds(0, n), :] \
            + yr_ref[...] + b1_ref[...]
        h = jnp.maximum(h, 0.0)
        rel_ref[pl.ds(0, n), :] = jnp.dot(h, wrel_ref[...],
                                          preferred_element_type=jnp.float32)
        root_ref[...] = jnp.dot(h, wroot_ref[...],
                                preferred_element_type=jnp.float32)

    return pl.pallas_call(
        body,
        out_shape=(
            jax.ShapeDtypeStruct((n_pad, _LANES), jnp.float32),
            jax.ShapeDtypeStruct((n, _LANES), jnp.float32),
        ),
    )(p1, y_root, b1, w2rel_t, w2root_t)


def _tc_out(p2, y_root, b2, c, n_pad):
    n = y_root.shape[0]

    def body(p2_ref, yr_ref, b2_ref, out_ref):
        o = p2_ref[0, pl.ds(0, n), :] + p2_ref[1, pl.ds(0, n), :] \
            + yr_ref[...] + b2_ref[...]
        col = lax.broadcasted_iota(jnp.int32, o.shape, 1)
        mask = col < c
        neg = jnp.float32(-1e30)
        om = jnp.where(mask, o, neg)
        m = jnp.max(om, axis=1, keepdims=True)
        e = jnp.where(mask, jnp.exp(o - m), 0.0)
        s = jnp.sum(e, axis=1, keepdims=True)
        out_ref[...] = ((o - m) - jnp.log(s))[:, :c]

    return pl.pallas_call(
        body,
        out_shape=jax.ShapeDtypeStruct((n, c), jnp.float32),
    )(p2, y_root, b2)


def kernel(x, edge_index, W1_rel, b1_rel, W1_root, W2_rel, b2_rel, W2_root):
    n, d = x.shape
    h_dim = W1_rel.shape[0]
    c = W2_rel.shape[0]
    e = edge_index.shape[1]

    kch = -(-e // (_NW * _CHUNK))
    e_pad = _NW * kch * _CHUNK
    n_pad = -(-(n + 1) // 128) * 128

    pad = e_pad - e
    if pad:
        src = jnp.concatenate(
            [edge_index[0], jnp.zeros((pad,), jnp.int32)])
        trash = n + jnp.arange(pad, dtype=jnp.int32) % (n_pad - n)
        dst = jnp.concatenate([edge_index[1], trash])
        srcp = src.reshape(_NW, kch, _CHUNK)
        dstp = dst.reshape(_NW, kch, _CHUNK)

        def seg_sum(y, zeros):
            return _sc_segment_sum(y, srcp, dstp, zeros, n_pad, kch)
    else:
        def seg_sum(y, zeros):
            return _sc_segment_sum_direct(y, edge_index, zeros, n_pad, kch)

    zeros = jnp.zeros((n_pad, _LANES), jnp.float32)

    w1 = jnp.concatenate([W1_rel, W1_root], axis=0).T
    y_rel, y_root = _tc_input(x, w1, n_pad)

    p1 = seg_sum(y_rel, zeros)

    w2rel_t = jnp.zeros((h_dim, _LANES), jnp.float32).at[:, :c].set(W2_rel.T)
    w2root_t = jnp.zeros((h_dim, _LANES), jnp.float32).at[:, :c].set(W2_root.T)
    b1 = b1_rel.reshape(1, h_dim)
    y2_rel, y2_root = _tc_mid(p1, y_root, b1, w2rel_t, w2root_t, n_pad)

    p2 = seg_sum(y2_rel, zeros)

    b2 = jnp.zeros((1, _LANES), jnp.float32).at[0, :c].set(b2_rel)
    return _tc_out(p2, y2_root, b2, c, n_pad)
